# baseline (speedup 1.0000x reference)
import jax
import jax.numpy as jnp
from jax import lax
from jax.experimental import pallas as pl
from jax.experimental.pallas import tpu as pltpu
from jax.experimental.pallas import tpu_sc as plsc

D = 128
B = 4096
K = 32
H = 2
D4 = 32
N_ITER = 30

_SELU_ALPHA = 1.6732632423543772
_SELU_SCALE = 1.0507009873554805

_NC, _NS = 2, 16
_NW = _NC * _NS
_NIDX = B * K + B
_RPW = _NIDX // _NW
_CH = 352
_NCHUNK = _RPW // _CH


def _sc_gather_body(table, idx_hbm, out_hbm, idx_v, buf, sem):
    wid = lax.axis_index("s") * _NC + lax.axis_index("c")
    base = wid * _RPW
    pltpu.sync_copy(idx_hbm.at[pl.ds(base, _RPW)], idx_v)
    for j in range(_NCHUNK):
        pltpu.async_copy(table.at[idx_v.at[pl.ds(j * _CH, _CH)]], buf, sem).wait()
        pltpu.sync_copy(buf, out_hbm.at[pl.ds(base + j * _CH, _CH)])


def _sc_gather(u2e, idx):
    f = pl.kernel(
        _sc_gather_body,
        mesh=plsc.VectorSubcoreMesh(core_axis_name="c", subcore_axis_name="s"),
        out_type=jax.ShapeDtypeStruct((_NIDX, D), jnp.float32),
        scratch_types=[
            pltpu.VMEM((_RPW,), jnp.int32),
            pltpu.VMEM((_CH, D), jnp.float32),
            pltpu.SemaphoreType.DMA,
        ],
    )
    return f(u2e, idx)


_TB = 256


def _normalize_rows(x):
    n = jnp.sqrt(jnp.sum(x * x, axis=-1, keepdims=True))
    return x / jnp.maximum(n, 1e-12)


def _selu(x):
    return _SELU_SCALE * jnp.where(x > 0, x, _SELU_ALPHA * (jnp.exp(x) - 1.0))


def _safe_pow(t, inv):
    pos = t > 0.0
    lg = jnp.log(jnp.where(pos, t, 1.0))
    return jnp.where(pos, jnp.exp(inv * lg), 0.0)


def _entmax(x, alpha):
    am1 = alpha - 1.0
    xs = x * am1
    inv = 1.0 / am1
    mx = jnp.max(xs, axis=1, keepdims=True)
    lo = mx - 1.0
    hi = mx
    for _ in range(N_ITER):
        mid = 0.5 * (lo + hi)
        f = jnp.sum(_safe_pow(jnp.maximum(xs - mid, 0.0), inv), axis=1,
                    keepdims=True) - 1.0
        ge = f >= 0.0
        lo = jnp.where(ge, mid, lo)
        hi = jnp.where(ge, hi, mid)
    tau = 0.5 * (lo + hi)
    p = _safe_pow(jnp.maximum(xs - tau, 0.0), inv)
    return p / jnp.sum(p, axis=1, keepdims=True)


def _attn_body(e_ref, s_ref, w1_ref, b1_ref, w2_ref, b2_ref, w3t_ref, b3_ref,
               l1t_ref, l1b_ref, out_ref):
    e = e_ref[...]
    en = _normalize_rows(e)
    en3 = en.reshape(_TB, K, D)
    u = s_ref[...]
    acc = jnp.zeros((_TB, D), jnp.float32)
    for h in range(H):
        u_n = _normalize_rows(u)
        a_e = jnp.dot(en, w1_ref[h, :D, :], preferred_element_type=jnp.float32)
        a_u = jnp.dot(u_n, w1_ref[h, D:, :], preferred_element_type=jnp.float32)
        b1 = b1_ref[h:h + 1, :].reshape(1, 1, D)
        a1 = _selu(a_e.reshape(_TB, K, D) + a_u[:, None, :] + b1)
        a2 = _selu(jnp.dot(a1.reshape(_TB * K, D), w2_ref[h],
                           preferred_element_type=jnp.float32)
                   + b2_ref[h:h + 1, :])
        w3 = w3t_ref[h].reshape(1, 1, D4)
        s = jnp.sum(a2.reshape(_TB, K, D4) * w3, axis=-1) + b3_ref[h:h + 1, 0:1]
        l1 = l1t_ref[h].reshape(1, 1, D)
        walpha = jax.nn.sigmoid(jnp.sum(en3 * l1, axis=-1)
                                + l1b_ref[h:h + 1, 0:1]) + 1.0
        att = _entmax(s, walpha)
        u = jnp.sum(en3 * att[:, :, None], axis=1)
        acc = acc + u
    out_ref[...] = acc * (1.0 / H)


def _full_spec(shape):
    return pl.BlockSpec(shape, lambda i: tuple(0 for _ in shape))


def _attn(e_flat, self_feats, att1_W, att1_b, att2_W, att2_b, att3_Wt, att3_b,
          lin1_Wt, lin1_b):
    return pl.pallas_call(
        _attn_body,
        grid=(B // _TB,),
        in_specs=[
            pl.BlockSpec((_TB * K, D), lambda i: (i, 0)),
            pl.BlockSpec((_TB, D), lambda i: (i, 0)),
            _full_spec((H, 2 * D, D)),
            _full_spec((H, D)),
            _full_spec((H, D, D4)),
            _full_spec((H, D4)),
            _full_spec((H, 1, D4)),
            _full_spec((H, 1)),
            _full_spec((H, 1, D)),
            _full_spec((H, 1)),
        ],
        out_specs=pl.BlockSpec((_TB, D), lambda i: (i, 0)),
        out_shape=jax.ShapeDtypeStruct((B, D), jnp.float32),
    )(e_flat, self_feats, att1_W, att1_b, att2_W, att2_b, att3_Wt, att3_b,
      lin1_Wt, lin1_b)


def _head_body(nf_ref, sf_ref, inw_ref, inb_ref, outw_ref, outb_ref, gw_ref,
               gb_ref, bng_ref, bnb_ref, bn1g_ref, bn1b_ref, o_ref):
    nf = nf_ref[...]
    sf = sf_ref[...]
    mu = jnp.mean(nf, axis=0, keepdims=True)
    xc = nf - mu
    var = jnp.mean(xc * xc, axis=0, keepdims=True)
    h0 = xc / jnp.sqrt(var + 1e-5) * bng_ref[...] + bnb_ref[...]
    h1 = _selu(jnp.dot(h0, inw_ref[...], preferred_element_type=jnp.float32)
               + inb_ref[...])
    mu1 = jnp.mean(h1, axis=0, keepdims=True)
    xc1 = h1 - mu1
    var1 = jnp.mean(xc1 * xc1, axis=0, keepdims=True)
    h1n = xc1 / jnp.sqrt(var1 + 1e-5) * bn1g_ref[...] + bn1b_ref[...]
    no = jnp.dot(h1n, outw_ref[...], preferred_element_type=jnp.float32) \
        + outb_ref[...]
    z = (jnp.dot(sf, gw_ref[0:D, :], preferred_element_type=jnp.float32)
         + jnp.dot(no, gw_ref[D:2 * D, :], preferred_element_type=jnp.float32)
         + jnp.dot(sf * no, gw_ref[2 * D:, :],
                   preferred_element_type=jnp.float32)
         + gb_ref[...])
    gama = jax.nn.sigmoid(z)
    o_ref[...] = gama * sf + (1.0 - gama) * no


def _head(nf, sf, in_W, in_b, out_W, out_b, gate_W, gate_b, bn_g, bn_b,
          bn1_g, bn1_b):
    return pl.pallas_call(
        _head_body,
        out_shape=jax.ShapeDtypeStruct((B, D), jnp.float32),
    )(nf, sf, in_W, in_b, out_W, out_b, gate_W, gate_b, bn_g, bn_b,
      bn1_g, bn1_b)


def kernel(nodes, neighbors, u2e, att1_W, att1_b, att2_W, att2_b, att3_W,
           att3_b, lin1_W, lin1_b, gate_W, gate_b, bn_g, bn_b, in_W, in_b,
           bn1_g, bn1_b, out_W, out_b):
    idx = jnp.concatenate([neighbors.reshape(-1).astype(jnp.int32),
                           nodes.astype(jnp.int32)])
    g = _sc_gather(u2e, idx)
    e_flat = g[:B * K]
    sf = g[B * K:]
    att3_Wt = jnp.transpose(att3_W, (0, 2, 1))
    lin1_Wt = jnp.transpose(lin1_W, (0, 2, 1))
    nf = _attn(e_flat, sf, att1_W, att1_b, att2_W, att2_b, att3_Wt, att3_b,
               lin1_Wt, lin1_b)
    return _head(nf, sf, in_W, in_b.reshape(1, D), out_W, out_b.reshape(1, D),
                 gate_W, gate_b.reshape(1, D), bn_g.reshape(1, D),
                 bn_b.reshape(1, D), bn1_g.reshape(1, D), bn1_b.reshape(1, D))
